# Initial kernel scaffold; baseline (speedup 1.0000x reference)
import jax
import jax.numpy as jnp
from jax.experimental import pallas as pl


def kernel(input_ids, embedding_weight):
    raise NotImplementedError("write your pallas kernel here")


# speedup vs baseline: 9.2385x; 9.2385x over previous
import functools

import jax
import jax.numpy as jnp
from jax import lax
from jax.experimental import pallas as pl
from jax.experimental.pallas import tpu as pltpu
from jax.experimental.pallas import tpu_sc as plsc

D = 128
C = 128


@functools.lru_cache(maxsize=None)
def _emb_call(B: int):
    info = plsc.get_sparse_core_info()
    NC, NS = info.num_cores, info.num_subcores
    NW = NC * NS
    b_per_w = B // NW
    n_chunks = b_per_w // C
    assert n_chunks % 2 == 0
    mesh = plsc.VectorSubcoreMesh(core_axis_name="c", subcore_axis_name="s")

    @functools.partial(
        pl.kernel,
        mesh=mesh,
        out_type=jax.ShapeDtypeStruct((B, D), jnp.float32),
        scratch_types=[
            pltpu.VMEM((n_chunks, C), jnp.int32),
            pltpu.VMEM((C, D), jnp.float32),
            pltpu.VMEM((C, D), jnp.float32),
            pltpu.SemaphoreType.DMA,
            pltpu.SemaphoreType.DMA,
        ],
    )
    def emb(idx_hbm, table_hbm, out_hbm, idx_v, buf0, buf1, sem0, sem1):
        wid = lax.axis_index("s") * NC + lax.axis_index("c")
        base = wid * b_per_w
        pltpu.sync_copy(idx_hbm.at[pl.ds(wid * n_chunks, n_chunks)], idx_v)
        bufs = (buf0, buf1)
        sems = (sem0, sem1)

        def start(g, b):
            pltpu.async_copy(table_hbm.at[idx_v.at[g]], bufs[b], sems[b])

        def wait(b):
            pltpu.make_async_copy(table_hbm.at[pl.ds(0, C)], bufs[b], sems[b]).wait()

        def copy_out(g, b):
            pltpu.sync_copy(bufs[b], out_hbm.at[pl.ds(base + g * C, C)])

        start(0, 0)

        def body(i, carry):
            for b in range(2):
                g = 2 * i + b
                nb = (b + 1) % 2

                @pl.when(g + 1 < n_chunks)
                def _():
                    start(g + 1, nb)

                wait(b)
                copy_out(g, b)
            return carry

        lax.fori_loop(0, n_chunks // 2, body, 0)

    return emb


def kernel(input_ids, embedding_weight):
    bt, h = input_ids.shape
    B = bt * h
    idx = input_ids.reshape(B // C, C).astype(jnp.int32)
    out = _emb_call(B)(idx, embedding_weight)
    return out.reshape(bt, h, D)
